# baseline (speedup 1.0000x reference)
import functools

import jax
import jax.numpy as jnp
from jax import lax
from jax.experimental import pallas as pl
from jax.experimental.pallas import tpu as pltpu
from jax.experimental.pallas import tpu_sc as plsc

EMBED_DIM = 64
NUM_CORES = 2
NUM_SUBCORES = 16
NUM_WORKERS = NUM_CORES * NUM_SUBCORES

B_TOTAL = 4096 * 200
B_PER_W = B_TOTAL // NUM_WORKERS
CHUNK = 512
N_CHUNKS = B_PER_W // CHUNK

_mesh = plsc.VectorSubcoreMesh(core_axis_name="c", subcore_axis_name="s")


@functools.partial(
    pl.kernel,
    mesh=_mesh,
    out_type=jax.ShapeDtypeStruct((B_TOTAL, EMBED_DIM), jnp.float32),
    compiler_params=pltpu.CompilerParams(use_tc_tiling_on_sc=False),
    scratch_types=[
        pltpu.VMEM((CHUNK,), jnp.int32),
        pltpu.VMEM((CHUNK, EMBED_DIM), jnp.float32),
        pltpu.SemaphoreType.DMA,
    ],
)
def _gather_kernel(idx_hbm, table_hbm, out_hbm, idx_v, rows_v, sem):
    wid = lax.axis_index("s") * NUM_CORES + lax.axis_index("c")
    base = wid * B_PER_W

    def body(i, _):
        off = base + i * CHUNK
        pltpu.sync_copy(idx_hbm.at[pl.ds(off, CHUNK)], idx_v)
        pltpu.async_copy(table_hbm.at[idx_v], rows_v, sem).wait()
        pltpu.sync_copy(rows_v, out_hbm.at[pl.ds(off, CHUNK)])
        return 0

    lax.fori_loop(0, N_CHUNKS, body, 0)


def kernel(input, table):
    idx = input.reshape(-1).astype(jnp.int32)
    out = _gather_kernel(idx, table)
    return out.reshape(input.shape + (EMBED_DIM,))


# speedup vs baseline: 1.0452x; 1.0452x over previous
import functools

import jax
import jax.numpy as jnp
from jax import lax
from jax.experimental import pallas as pl
from jax.experimental.pallas import tpu as pltpu
from jax.experimental.pallas import tpu_sc as plsc

EMBED_DIM = 64
NUM_CORES = 2
NUM_SUBCORES = 16
NUM_WORKERS = NUM_CORES * NUM_SUBCORES

B_TOTAL = 4096 * 200
B_PER_W = B_TOTAL // NUM_WORKERS
CHUNK = 512
N_CHUNKS = B_PER_W // CHUNK
NBUF = 3

_mesh = plsc.VectorSubcoreMesh(core_axis_name="c", subcore_axis_name="s")

_scratch = (
    [pltpu.VMEM((B_PER_W,), jnp.int32)]
    + [pltpu.VMEM((CHUNK, EMBED_DIM), jnp.float32) for _ in range(NBUF)]
    + [pltpu.SemaphoreType.DMA for _ in range(2 * NBUF)]
)


@functools.partial(
    pl.kernel,
    mesh=_mesh,
    out_type=jax.ShapeDtypeStruct((B_TOTAL, EMBED_DIM), jnp.float32),
    compiler_params=pltpu.CompilerParams(use_tc_tiling_on_sc=False),
    scratch_types=_scratch,
)
def _gather_kernel(idx_hbm, table_hbm, out_hbm, idx_all, *bufs_sems):
    row_bufs = bufs_sems[:NBUF]
    gsems = bufs_sems[NBUF:2 * NBUF]
    osems = bufs_sems[2 * NBUF:]

    wid = lax.axis_index("s") * NUM_CORES + lax.axis_index("c")
    base = wid * B_PER_W

    pltpu.sync_copy(idx_hbm.at[pl.ds(base, B_PER_W)], idx_all)

    def gather_start(i, b):
        return pltpu.async_copy(
            table_hbm.at[idx_all.at[pl.ds(i * CHUNK, CHUNK)]],
            row_bufs[b], gsems[b])

    def out_start(i, b):
        return pltpu.async_copy(
            row_bufs[b], out_hbm.at[pl.ds(base + i * CHUNK, CHUNK)],
            osems[b])

    def gather_wait(b):
        pltpu.make_async_copy(
            out_hbm.at[pl.ds(base, CHUNK)], row_bufs[b], gsems[b]).wait()

    def out_wait(b):
        pltpu.make_async_copy(
            out_hbm.at[pl.ds(base, CHUNK)], row_bufs[b], osems[b]).wait()

    for b in range(NBUF):
        gather_start(b, b)

    def body(g, _):
        for b in range(NBUF):
            i = g * NBUF + b

            @pl.when(i < N_CHUNKS)
            def _():
                gather_wait(b)
                out_start(i, b)
                nxt = i + NBUF

                @pl.when(nxt < N_CHUNKS)
                def _():
                    out_wait(b)
                    gather_start(nxt, b)
        return 0

    n_outer = (N_CHUNKS + NBUF - 1) // NBUF
    lax.fori_loop(0, n_outer, body, 0)

    for b in range(NBUF):
        out_wait(b)


def kernel(input, table):
    idx = input.reshape(-1).astype(jnp.int32)
    out = _gather_kernel(idx, table)
    return out.reshape(input.shape + (EMBED_DIM,))
